# Initial kernel scaffold; baseline (speedup 1.0000x reference)
import jax
import jax.numpy as jnp
from jax.experimental import pallas as pl


def kernel(W, b, b2d, res_with_dim):
    raise NotImplementedError("write your pallas kernel here")


# speedup vs baseline: 4.1122x; 4.1122x over previous
import jax
import jax.numpy as jnp
from jax.experimental import pallas as pl
from jax.experimental.pallas import tpu as pltpu


def _decoder_body(x_ref, wt_ref, b_ref, o_ref):
    x = x_ref[0].astype(jnp.bfloat16)
    y = jax.lax.dot_general(
        x, wt_ref[...],
        dimension_numbers=(((0,), (0,)), ((), ())),
        preferred_element_type=jnp.float32,
    )
    o_ref[0] = y + b_ref[...]


def _decoder_apply(W, b2d, res_with_dim):
    B, C, T = res_with_dim.shape
    c_out = W.shape[0]
    wt = W.T.astype(jnp.bfloat16)

    cost = pl.CostEstimate(
        flops=2 * B * T * C * c_out,
        transcendentals=0,
        bytes_accessed=4 * (B * C * T + B * T * c_out + c_out) + 2 * C * c_out,
    )
    return pl.pallas_call(
        _decoder_body,
        out_shape=jax.ShapeDtypeStruct((B, T, c_out), jnp.float32),
        grid=(B,),
        in_specs=[
            pl.BlockSpec((1, C, T), lambda bi: (bi, 0, 0)),
            pl.BlockSpec((C, c_out), lambda bi: (0, 0)),
            pl.BlockSpec((1, c_out), lambda bi: (0, 0)),
        ],
        out_specs=pl.BlockSpec((1, T, c_out), lambda bi: (bi, 0, 0)),
        compiler_params=pltpu.CompilerParams(
            dimension_semantics=("parallel",),
            vmem_limit_bytes=64 * 1024 * 1024,
        ),
        cost_estimate=cost,
    )(res_with_dim, wt, b2d)


def kernel(W, b, b2d, res_with_dim):
    out = _decoder_apply(W, b2d, res_with_dim)
    return {"out": out, "memory_adj": None, "adj": None, "attn": None}


# speedup vs baseline: 5.7753x; 1.4044x over previous
import jax
import jax.numpy as jnp
from jax.experimental import pallas as pl
from jax.experimental.pallas import tpu as pltpu


_BB = 2


def _decoder_body(x_ref, w_ref, b_ref, o_ref):
    w = w_ref[...].astype(jnp.bfloat16)
    bias = b_ref[...]
    for bb in range(_BB):
        x = x_ref[bb].astype(jnp.bfloat16)
        y = jax.lax.dot_general(
            x, w,
            dimension_numbers=(((0,), (1,)), ((), ())),
            preferred_element_type=jnp.float32,
        )
        o_ref[bb] = y + bias


def _decoder_apply(W, b2d, res_with_dim):
    B, C, T = res_with_dim.shape
    c_out = W.shape[0]

    cost = pl.CostEstimate(
        flops=2 * B * T * C * c_out,
        transcendentals=0,
        bytes_accessed=4 * (B * C * T + B * T * c_out + c_out + C * c_out),
    )
    return pl.pallas_call(
        _decoder_body,
        out_shape=jax.ShapeDtypeStruct((B, T, c_out), jnp.float32),
        grid=(B // _BB,),
        in_specs=[
            pl.BlockSpec((_BB, C, T), lambda bi: (bi, 0, 0)),
            pl.BlockSpec((c_out, C), lambda bi: (0, 0)),
            pl.BlockSpec((1, c_out), lambda bi: (0, 0)),
        ],
        out_specs=pl.BlockSpec((_BB, T, c_out), lambda bi: (bi, 0, 0)),
        compiler_params=pltpu.CompilerParams(
            dimension_semantics=("parallel",),
            vmem_limit_bytes=64 * 1024 * 1024,
        ),
        cost_estimate=cost,
    )(res_with_dim, W, b2d)


def kernel(W, b, b2d, res_with_dim):
    out = _decoder_apply(W, b2d, res_with_dim)
    return {"out": out, "memory_adj": None, "adj": None, "attn": None}


# speedup vs baseline: 6.6764x; 1.1560x over previous
import jax
import jax.numpy as jnp
from jax.experimental import pallas as pl
from jax.experimental.pallas import tpu as pltpu


_BB = 4


def _decoder_body(x_ref, w_ref, b_ref, o_ref):
    w = w_ref[...].astype(jnp.bfloat16)
    bias = b_ref[...]
    for bb in range(_BB):
        x = x_ref[bb].astype(jnp.bfloat16)
        y = jax.lax.dot_general(
            x, w,
            dimension_numbers=(((0,), (1,)), ((), ())),
            preferred_element_type=jnp.float32,
        )
        o_ref[bb] = y + bias


def _decoder_apply(W, b2d, res_with_dim):
    B, C, T = res_with_dim.shape
    c_out = W.shape[0]

    cost = pl.CostEstimate(
        flops=2 * B * T * C * c_out,
        transcendentals=0,
        bytes_accessed=4 * (B * C * T + B * T * c_out + c_out + C * c_out),
    )
    return pl.pallas_call(
        _decoder_body,
        out_shape=jax.ShapeDtypeStruct((B, T, c_out), jnp.float32),
        grid=(B // _BB,),
        in_specs=[
            pl.BlockSpec((_BB, C, T), lambda bi: (bi, 0, 0)),
            pl.BlockSpec((c_out, C), lambda bi: (0, 0)),
            pl.BlockSpec((1, c_out), lambda bi: (0, 0)),
        ],
        out_specs=pl.BlockSpec((_BB, T, c_out), lambda bi: (bi, 0, 0)),
        compiler_params=pltpu.CompilerParams(
            dimension_semantics=("parallel",),
            vmem_limit_bytes=64 * 1024 * 1024,
        ),
        cost_estimate=cost,
    )(res_with_dim, W, b2d)


def kernel(W, b, b2d, res_with_dim):
    out = _decoder_apply(W, b2d, res_with_dim)
    return {"out": out, "memory_adj": None, "adj": None, "attn": None}


# speedup vs baseline: 6.8304x; 1.0231x over previous
import jax
import jax.numpy as jnp
from jax.experimental import pallas as pl
from jax.experimental.pallas import tpu as pltpu


_BB = 8


def _decoder_body(x_ref, w_ref, b_ref, o_ref):
    w = w_ref[...].astype(jnp.bfloat16)
    bias = b_ref[...]
    for bb in range(_BB):
        x = x_ref[bb].astype(jnp.bfloat16)
        y = jax.lax.dot_general(
            x, w,
            dimension_numbers=(((0,), (1,)), ((), ())),
            preferred_element_type=jnp.float32,
        )
        o_ref[bb] = y + bias


def _decoder_apply(W, b2d, res_with_dim):
    B, C, T = res_with_dim.shape
    c_out = W.shape[0]

    cost = pl.CostEstimate(
        flops=2 * B * T * C * c_out,
        transcendentals=0,
        bytes_accessed=4 * (B * C * T + B * T * c_out + c_out + C * c_out),
    )
    return pl.pallas_call(
        _decoder_body,
        out_shape=jax.ShapeDtypeStruct((B, T, c_out), jnp.float32),
        grid=(B // _BB,),
        in_specs=[
            pl.BlockSpec((_BB, C, T), lambda bi: (bi, 0, 0)),
            pl.BlockSpec((c_out, C), lambda bi: (0, 0)),
            pl.BlockSpec((1, c_out), lambda bi: (0, 0)),
        ],
        out_specs=pl.BlockSpec((_BB, T, c_out), lambda bi: (bi, 0, 0)),
        compiler_params=pltpu.CompilerParams(
            dimension_semantics=("parallel",),
            vmem_limit_bytes=64 * 1024 * 1024,
        ),
        cost_estimate=cost,
    )(res_with_dim, W, b2d)


def kernel(W, b, b2d, res_with_dim):
    out = _decoder_apply(W, b2d, res_with_dim)
    return {"out": out, "memory_adj": None, "adj": None, "attn": None}


# speedup vs baseline: 7.2400x; 1.0600x over previous
import jax
import jax.numpy as jnp
from jax.experimental import pallas as pl
from jax.experimental.pallas import tpu as pltpu

_DEPTH_IN = 6
_DEPTH_OUT = 4


def _make_body(B):
    def _body(x_hbm, w_ref, b_ref, o_hbm, xbuf, obuf, in_sems, out_sems):
        w = w_ref[...].astype(jnp.bfloat16)
        bias = b_ref[...]

        for i in range(min(_DEPTH_IN, B)):
            pltpu.make_async_copy(
                x_hbm.at[i], xbuf.at[i], in_sems.at[i]).start()

        for s in range(B):
            si = s % _DEPTH_IN
            so = s % _DEPTH_OUT
            pltpu.make_async_copy(
                x_hbm.at[s], xbuf.at[si], in_sems.at[si]).wait()
            if s >= _DEPTH_OUT:
                pltpu.make_async_copy(
                    obuf.at[so], o_hbm.at[s - _DEPTH_OUT],
                    out_sems.at[so]).wait()
            x = xbuf[si].astype(jnp.bfloat16)
            y = jax.lax.dot_general(
                x, w,
                dimension_numbers=(((0,), (1,)), ((), ())),
                preferred_element_type=jnp.float32,
            )
            obuf[so] = y + bias
            pltpu.make_async_copy(
                obuf.at[so], o_hbm.at[s], out_sems.at[so]).start()
            nxt = s + _DEPTH_IN
            if nxt < B:
                pltpu.make_async_copy(
                    x_hbm.at[nxt], xbuf.at[nxt % _DEPTH_IN],
                    in_sems.at[nxt % _DEPTH_IN]).start()

        for s in range(max(B - _DEPTH_OUT, 0), B):
            so = s % _DEPTH_OUT
            pltpu.make_async_copy(
                obuf.at[so], o_hbm.at[s], out_sems.at[so]).wait()
    return _body


def _decoder_apply(W, b2d, res_with_dim):
    B, C, T = res_with_dim.shape
    c_out = W.shape[0]

    cost = pl.CostEstimate(
        flops=2 * B * T * C * c_out,
        transcendentals=0,
        bytes_accessed=4 * (B * C * T + B * T * c_out + c_out + C * c_out),
    )
    return pl.pallas_call(
        _make_body(B),
        out_shape=jax.ShapeDtypeStruct((B, T, c_out), jnp.float32),
        in_specs=[
            pl.BlockSpec(memory_space=pl.ANY),
            pl.BlockSpec((c_out, C), lambda: (0, 0)),
            pl.BlockSpec((1, c_out), lambda: (0, 0)),
        ],
        out_specs=pl.BlockSpec(memory_space=pl.ANY),
        scratch_shapes=[
            pltpu.VMEM((_DEPTH_IN, C, T), jnp.float32),
            pltpu.VMEM((_DEPTH_OUT, T, c_out), jnp.float32),
            pltpu.SemaphoreType.DMA((_DEPTH_IN,)),
            pltpu.SemaphoreType.DMA((_DEPTH_OUT,)),
        ],
        compiler_params=pltpu.CompilerParams(
            vmem_limit_bytes=64 * 1024 * 1024,
        ),
        cost_estimate=cost,
    )(res_with_dim, W, b2d)


def kernel(W, b, b2d, res_with_dim):
    out = _decoder_apply(W, b2d, res_with_dim)
    return {"out": out, "memory_adj": None, "adj": None, "attn": None}
